# baseline (speedup 1.0000x reference)
import functools

import jax
import jax.numpy as jnp
from jax import lax
from jax.experimental import pallas as pl
from jax.experimental.pallas import tpu as pltpu
from jax.experimental.pallas import tpu_sc as plsc

_LANES = 16
_NTILES = 32
_BLOCK = 128


@functools.lru_cache(maxsize=None)
def _make_kernels(n_rows: int, n_cols: int):
    per_tile = n_rows // _NTILES
    assert per_tile == _LANES
    chunks = n_rows // _LANES

    mesh = plsc.VectorSubcoreMesh(core_axis_name="c", subcore_axis_name="s")
    params = pltpu.CompilerParams(needs_layout_passes=False)

    @functools.partial(
        pl.kernel,
        mesh=mesh,
        compiler_params=params,
        out_type=jax.ShapeDtypeStruct((n_rows,), jnp.float32),
        scratch_types=[
            pltpu.VMEM((_LANES,), jnp.int32),
            pltpu.VMEM((_LANES,), jnp.int32),
            pltpu.VMEM((_LANES, _BLOCK), jnp.float32),
            pltpu.VMEM((_LANES,), jnp.float32),
            pltpu.VMEM((_LANES,), jnp.float32),
            pltpu.SemaphoreType.DMA,
        ],
    )
    def _gather(prob_hbm, tgt_hbm, rew_hbm, out_hbm,
                blk_v, off_v, val_v, rew_v, acc_v, sem):
        wid = lax.axis_index("s") * 2 + lax.axis_index("c")
        base = wid * _LANES
        pltpu.sync_copy(tgt_hbm.at[pl.ds(base, _LANES)], blk_v)
        pltpu.sync_copy(rew_hbm.at[pl.ds(base, _LANES)], rew_v)
        rows = base + lax.broadcasted_iota(jnp.int32, (_LANES,), 0)
        flat = blk_v[...] + rows * n_cols
        off_v[...] = flat & (_BLOCK - 1)
        blk_v[...] = lax.shift_right_logical(flat, 7)
        pltpu.async_copy(prob_hbm.at[blk_v], val_v, sem).wait()
        rowsel = lax.broadcasted_iota(jnp.int32, (_LANES,), 0)
        picked = plsc.load_gather(val_v, [rowsel, off_v[...]])
        acc_v[...] = picked * rew_v[...]
        pltpu.sync_copy(acc_v, out_hbm.at[pl.ds(base, _LANES)])

    @functools.partial(
        pl.kernel,
        mesh=mesh,
        compiler_params=params,
        out_type=jax.ShapeDtypeStruct((_LANES,), jnp.float32),
        scratch_types=[
            pltpu.VMEM((n_rows,), jnp.float32),
            pltpu.VMEM((_LANES,), jnp.float32),
        ],
    )
    def _reduce(prod_hbm, out_hbm, prod_v, acc_v):
        cid = lax.axis_index("c")
        sid = lax.axis_index("s")

        @pl.when((cid == 0) & (sid == 0))
        def _():
            pltpu.sync_copy(prod_hbm, prod_v)
            total = jnp.zeros((_LANES,), jnp.float32)
            for k in range(chunks):
                total = total + prod_v[pl.ds(k * _LANES, _LANES)]
            lane = lax.broadcasted_iota(jnp.int32, (_LANES,), 0)
            for sh in (8, 4, 2, 1):
                acc_v[...] = total
                total = total + plsc.load_gather(
                    acc_v, [(lane + sh) & (_LANES - 1)])
            acc_v[...] = -total
            pltpu.sync_copy(acc_v, out_hbm)

    return _gather, _reduce


def kernel(prob, target, reward):
    n_rows = prob.shape[0] * prob.shape[1]
    n_cols = prob.shape[2]
    blocked = prob.reshape(n_rows * n_cols // _BLOCK, _BLOCK)
    tgt = target.astype(jnp.int32)
    rew = reward.astype(jnp.float32)
    gather, reduce = _make_kernels(n_rows, n_cols)
    prod = gather(blocked, tgt, rew)
    out = reduce(prod)
    return out[0]


# speedup vs baseline: 11.6034x; 11.6034x over previous
import functools

import jax
import jax.numpy as jnp
from jax import lax
from jax.experimental import pallas as pl
from jax.experimental.pallas import tpu as pltpu
from jax.experimental.pallas import tpu_sc as plsc

_LANES = 16
_NTILES = 32
_BLOCK = 128


@functools.lru_cache(maxsize=None)
def _make_kernels(n_rows: int, n_cols: int):
    per_tile = n_rows // _NTILES
    assert per_tile == _LANES
    chunks = n_rows // _LANES
    last_start = n_cols - _BLOCK

    mesh = plsc.VectorSubcoreMesh(core_axis_name="c", subcore_axis_name="s")
    params = pltpu.CompilerParams(needs_layout_passes=False)

    @functools.partial(
        pl.kernel,
        mesh=mesh,
        compiler_params=params,
        out_type=jax.ShapeDtypeStruct((n_rows,), jnp.float32),
        scratch_types=[
            pltpu.VMEM((_LANES,), jnp.int32),
            pltpu.VMEM((_LANES * 8, _BLOCK), jnp.float32),
            pltpu.VMEM((_LANES,), jnp.float32),
            pltpu.VMEM((_LANES,), jnp.float32),
            pltpu.SemaphoreType.DMA,
        ],
    )
    def _gather(prob_hbm, tgt_hbm, rew_hbm, out_hbm,
                tgt_v, val_v, rew_v, acc_v, sem):
        wid = lax.axis_index("s") * 2 + lax.axis_index("c")
        base = wid * _LANES
        pltpu.sync_copy(tgt_hbm.at[pl.ds(base, _LANES)], tgt_v)
        pltpu.sync_copy(rew_hbm.at[pl.ds(base, _LANES)], rew_v)
        iot = lax.broadcasted_iota(jnp.int32, (_LANES,), 0)
        winv = tgt_v[...] & -_BLOCK
        copies = []
        for j in range(_LANES):
            c0 = pl.multiple_of(
                jnp.sum(jnp.where(iot == j, winv, 0)), _BLOCK)
            r0 = base + 8 * (j // 8)
            copies.append(pltpu.async_copy(
                prob_hbm.at[pl.ds(r0, 8), pl.ds(c0, _BLOCK)],
                val_v.at[pl.ds(j * 8, 8)], sem))
        for cp in copies:
            cp.wait()
        lanes = tgt_v[...] & (_BLOCK - 1)
        rowsel = iot * 8 + (iot & 7)
        picked = plsc.load_gather(val_v, [rowsel, lanes])
        acc_v[...] = picked * rew_v[...]
        pltpu.sync_copy(acc_v, out_hbm.at[pl.ds(base, _LANES)])

    @functools.partial(
        pl.kernel,
        mesh=mesh,
        compiler_params=params,
        out_type=jax.ShapeDtypeStruct((_LANES,), jnp.float32),
        scratch_types=[
            pltpu.VMEM((n_rows,), jnp.float32),
            pltpu.VMEM((_LANES,), jnp.float32),
        ],
    )
    def _reduce(prod_hbm, out_hbm, prod_v, acc_v):
        cid = lax.axis_index("c")
        sid = lax.axis_index("s")

        @pl.when((cid == 0) & (sid == 0))
        def _():
            pltpu.sync_copy(prod_hbm, prod_v)
            total = jnp.zeros((_LANES,), jnp.float32)
            for k in range(chunks):
                total = total + prod_v[pl.ds(k * _LANES, _LANES)]
            lane = lax.broadcasted_iota(jnp.int32, (_LANES,), 0)
            for sh in (8, 4, 2, 1):
                acc_v[...] = total
                total = total + plsc.load_gather(
                    acc_v, [(lane + sh) & (_LANES - 1)])
            acc_v[...] = -total
            pltpu.sync_copy(acc_v, out_hbm)

    return _gather, _reduce


def kernel(prob, target, reward):
    n_rows = prob.shape[0] * prob.shape[1]
    n_cols = prob.shape[2]
    prob2 = prob.reshape(n_rows, n_cols)
    tgt = target.astype(jnp.int32)
    rew = reward.astype(jnp.float32)
    gather, reduce = _make_kernels(n_rows, n_cols)
    prod = gather(prob2, tgt, rew)
    out = reduce(prod)
    return out[0]


# speedup vs baseline: 13.1276x; 1.1314x over previous
import functools

import jax
import jax.numpy as jnp
from jax import lax
from jax.experimental import pallas as pl
from jax.experimental.pallas import tpu as pltpu
from jax.experimental.pallas import tpu_sc as plsc

_LANES = 16
_NTILES = 32
_BLOCK = 128


@functools.lru_cache(maxsize=None)
def _make_kernels(n_rows: int, n_cols: int):
    per_tile = n_rows // _NTILES
    assert per_tile == _LANES
    chunks = n_rows // _LANES
    last_start = n_cols - _BLOCK

    mesh = plsc.VectorSubcoreMesh(core_axis_name="c", subcore_axis_name="s")
    params = pltpu.CompilerParams(needs_layout_passes=False)

    @functools.partial(
        pl.kernel,
        mesh=mesh,
        compiler_params=params,
        out_type=jax.ShapeDtypeStruct((n_rows,), jnp.float32),
        scratch_types=[
            pltpu.VMEM((_LANES,), jnp.int32),
            pltpu.VMEM((_LANES * 8, _BLOCK), jnp.float32),
            pltpu.VMEM((_LANES,), jnp.float32),
            pltpu.VMEM((_LANES,), jnp.float32),
            pltpu.SemaphoreType.DMA,
        ],
    )
    def _gather(prob_hbm, tgt_hbm, rew_hbm, out_hbm,
                tgt_v, val_v, rew_v, acc_v, sem):
        wid = lax.axis_index("s") * 2 + lax.axis_index("c")
        base = wid * _LANES
        pltpu.sync_copy(tgt_hbm.at[pl.ds(base, _LANES)], tgt_v)
        pltpu.sync_copy(rew_hbm.at[pl.ds(base, _LANES)], rew_v)
        iot = lax.broadcasted_iota(jnp.int32, (_LANES,), 0)
        winv = tgt_v[...] & -_BLOCK
        copies = []
        for j in range(_LANES):
            c0 = pl.multiple_of(
                jnp.sum(jnp.where(iot == j, winv, 0)), _BLOCK)
            r0 = base + 8 * (j // 8)
            copies.append(pltpu.async_copy(
                prob_hbm.at[pl.ds(r0, 8), pl.ds(c0, _BLOCK)],
                val_v.at[pl.ds(j * 8, 8)], sem))
        for cp in copies:
            cp.wait()
        lanes = tgt_v[...] & (_BLOCK - 1)
        rowsel = iot * 8 + (iot & 7)
        picked = plsc.load_gather(val_v, [rowsel, lanes])
        acc_v[...] = picked * rew_v[...]
        pltpu.sync_copy(acc_v, out_hbm.at[pl.ds(base, _LANES)])

    def _reduce_body(prod_ref, out_ref):
        out_ref[0, 0] = -jnp.sum(prod_ref[...])

    _reduce = pl.pallas_call(
        _reduce_body,
        out_shape=jax.ShapeDtypeStruct((1, 1), jnp.float32),
        out_specs=pl.BlockSpec(memory_space=pltpu.SMEM),
    )

    return _gather, _reduce


def kernel(prob, target, reward):
    n_rows = prob.shape[0] * prob.shape[1]
    n_cols = prob.shape[2]
    prob2 = prob.reshape(n_rows, n_cols)
    tgt = target.astype(jnp.int32)
    rew = reward.astype(jnp.float32)
    gather, reduce = _make_kernels(n_rows, n_cols)
    prod = gather(prob2, tgt, rew)
    out = reduce(prod)
    return out[0, 0]


# speedup vs baseline: 13.4525x; 1.0247x over previous
import functools

import jax
import jax.numpy as jnp
from jax import lax
from jax.experimental import pallas as pl
from jax.experimental.pallas import tpu as pltpu
from jax.experimental.pallas import tpu_sc as plsc

_LANES = 16
_NTILES = 16
_BLOCK = 128


@functools.lru_cache(maxsize=None)
def _make_kernels(n_rows: int, n_cols: int):
    per_tile = n_rows // _NTILES
    t_chunks = per_tile // _LANES

    mesh = plsc.VectorSubcoreMesh(core_axis_name="c", subcore_axis_name="s", num_cores=1)
    params = pltpu.CompilerParams(needs_layout_passes=False)

    @functools.partial(
        pl.kernel,
        mesh=mesh,
        compiler_params=params,
        out_type=jax.ShapeDtypeStruct((n_rows,), jnp.float32),
        scratch_types=[
            pltpu.VMEM((per_tile,), jnp.int32),
            pltpu.VMEM((per_tile * 8, _BLOCK), jnp.float32),
            pltpu.VMEM((per_tile,), jnp.float32),
            pltpu.VMEM((per_tile,), jnp.float32),
            pltpu.SemaphoreType.DMA,
        ],
    )
    def _gather(prob_hbm, tgt_hbm, rew_hbm, out_hbm,
                tgt_v, val_v, rew_v, acc_v, sem):
        wid = lax.axis_index("s")
        base = wid * per_tile
        pltpu.sync_copy(tgt_hbm.at[pl.ds(base, per_tile)], tgt_v)
        pltpu.sync_copy(rew_hbm.at[pl.ds(base, per_tile)], rew_v)
        iot = lax.broadcasted_iota(jnp.int32, (_LANES,), 0)
        copies = []
        for k in range(t_chunks):
            winv = tgt_v[pl.ds(k * _LANES, _LANES)] & -_BLOCK
            for j in range(_LANES):
                row = k * _LANES + j
                c0 = pl.multiple_of(
                    jnp.sum(jnp.where(iot == j, winv, 0)), _BLOCK)
                r0 = base + 8 * (row // 8)
                copies.append(pltpu.async_copy(
                    prob_hbm.at[pl.ds(r0, 8), pl.ds(c0, _BLOCK)],
                    val_v.at[pl.ds(row * 8, 8)], sem))
        for cp in copies:
            cp.wait()
        for k in range(t_chunks):
            lanes = tgt_v[pl.ds(k * _LANES, _LANES)] & (_BLOCK - 1)
            rowsel = (k * _LANES + iot) * 8 + (iot & 7)
            picked = plsc.load_gather(val_v, [rowsel, lanes])
            acc_v[pl.ds(k * _LANES, _LANES)] = (
                picked * rew_v[pl.ds(k * _LANES, _LANES)])
        pltpu.sync_copy(acc_v, out_hbm.at[pl.ds(base, per_tile)])

    def _reduce_body(prod_ref, out_ref):
        out_ref[0, 0] = -jnp.sum(prod_ref[...])

    _reduce = pl.pallas_call(
        _reduce_body,
        out_shape=jax.ShapeDtypeStruct((1, 1), jnp.float32),
        out_specs=pl.BlockSpec(memory_space=pltpu.SMEM),
    )

    return _gather, _reduce


def kernel(prob, target, reward):
    n_rows = prob.shape[0] * prob.shape[1]
    n_cols = prob.shape[2]
    prob2 = prob.reshape(n_rows, n_cols)
    tgt = target.astype(jnp.int32)
    rew = reward.astype(jnp.float32)
    gather, reduce = _make_kernels(n_rows, n_cols)
    prod = gather(prob2, tgt, rew)
    out = reduce(prod)
    return out[0, 0]


# speedup vs baseline: 13.5572x; 1.0078x over previous
import functools

import jax
import jax.numpy as jnp
from jax import lax
from jax.experimental import pallas as pl
from jax.experimental.pallas import tpu as pltpu
from jax.experimental.pallas import tpu_sc as plsc

_LANES = 16
_NTILES = 16
_BLOCK = 128


@functools.lru_cache(maxsize=None)
def _make_kernels(n_rows: int, n_cols: int):
    per_tile = n_rows // _NTILES
    t_chunks = per_tile // _LANES

    mesh = plsc.VectorSubcoreMesh(core_axis_name="c", subcore_axis_name="s", num_cores=1)
    params = pltpu.CompilerParams(needs_layout_passes=False)

    @functools.partial(
        pl.kernel,
        mesh=mesh,
        compiler_params=params,
        out_type=jax.ShapeDtypeStruct((n_rows,), jnp.float32),
        scratch_types=[
            pltpu.VMEM((per_tile,), jnp.int32),
            pltpu.VMEM((per_tile * 8, _BLOCK), jnp.float32),
            pltpu.VMEM((per_tile,), jnp.float32),
            pltpu.VMEM((per_tile,), jnp.float32),
            pltpu.SemaphoreType.DMA,
        ],
    )
    def _gather(prob_hbm, tgt_hbm, rew_hbm, out_hbm,
                tgt_v, val_v, rew_v, acc_v, sem):
        wid = lax.axis_index("s")
        base = wid * per_tile
        pltpu.sync_copy(tgt_hbm.at[pl.ds(base, per_tile)], tgt_v)
        pltpu.sync_copy(rew_hbm.at[pl.ds(base, per_tile)], rew_v)
        iot = lax.broadcasted_iota(jnp.int32, (_LANES,), 0)
        for k in range(t_chunks):
            winv = tgt_v[pl.ds(k * _LANES, _LANES)] & -_BLOCK

            def _fire(j, _):
                row = k * _LANES + j
                c0 = pl.multiple_of(
                    jnp.sum(jnp.where(iot == j, winv, 0)), _BLOCK)
                r0 = base + pl.multiple_of((row >> 3) << 3, 8)
                pltpu.async_copy(
                    prob_hbm.at[pl.ds(r0, 8), pl.ds(c0, _BLOCK)],
                    val_v.at[pl.ds(pl.multiple_of(row * 8, 8), 8)], sem)
                return _

            lax.fori_loop(0, _LANES, _fire, 0)

        def _drain(j, _):
            pltpu.make_async_copy(
                prob_hbm.at[pl.ds(0, 8), pl.ds(0, _BLOCK)],
                val_v.at[pl.ds(0, 8)], sem).wait()
            return _

        lax.fori_loop(0, per_tile, _drain, 0)
        for k in range(t_chunks):
            lanes = tgt_v[pl.ds(k * _LANES, _LANES)] & (_BLOCK - 1)
            rowsel = (k * _LANES + iot) * 8 + (iot & 7)
            picked = plsc.load_gather(val_v, [rowsel, lanes])
            acc_v[pl.ds(k * _LANES, _LANES)] = (
                picked * rew_v[pl.ds(k * _LANES, _LANES)])
        pltpu.sync_copy(acc_v, out_hbm.at[pl.ds(base, per_tile)])

    def _reduce_body(prod_ref, out_ref):
        out_ref[0, 0] = -jnp.sum(prod_ref[...])

    _reduce = pl.pallas_call(
        _reduce_body,
        out_shape=jax.ShapeDtypeStruct((1, 1), jnp.float32),
        out_specs=pl.BlockSpec(memory_space=pltpu.SMEM),
    )

    return _gather, _reduce


def kernel(prob, target, reward):
    n_rows = prob.shape[0] * prob.shape[1]
    n_cols = prob.shape[2]
    prob2 = prob.reshape(n_rows, n_cols)
    tgt = target.astype(jnp.int32)
    rew = reward.astype(jnp.float32)
    gather, reduce = _make_kernels(n_rows, n_cols)
    prod = gather(prob2, tgt, rew)
    out = reduce(prod)
    return out[0, 0]
